# baseline (speedup 1.0000x reference)
import functools

import jax
import jax.numpy as jnp
from jax import lax
from jax.experimental import pallas as pl
from jax.experimental.pallas import tpu as pltpu
from jax.experimental.pallas import tpu_sc as plsc

N = 10000
E = 160000
D_IN = 256
HID = 256
OUT = 128
REL = 8
G = 64

NSUB = 16
HALF = 128
EPT = E // NSUB
CHUNK = 80
NCHUNK = EPT // CHUNK
ROWS_PT = N // NSUB
GEDGE = 25

_mesh = plsc.VectorSubcoreMesh(core_axis_name="c", subcore_axis_name="s")


def _scale_body(et_h, src_h, dst_h, gidx_h, scale_h,
                cnt, etb, srcb, dstb, binb, gb, ones, cbuf, sbuf, zb,
                sem_h, sem_g, sem_st, sem_w):
    c = lax.axis_index("c")
    s = lax.axis_index("s")
    z16 = jnp.zeros((16,), jnp.float32)
    one16 = jnp.full((16,), 1.0, jnp.float32)

    def _zrow(i, carry):
        zb[i, :] = z16
        return carry
    lax.fori_loop(0, 250, _zrow, 0)
    for k in range(20):
        pltpu.sync_copy(zb, cnt.at[pl.ds(s * 5000 + k * 250, 250)])

    def _orow(i, carry):
        ones[i, :] = one16
        return carry
    lax.fori_loop(0, CHUNK, _orow, 0)
    plsc.subcore_barrier()

    def _grp(g, carry):
        pltpu.sync_copy(et_h.at[s, pl.ds(g * GEDGE, GEDGE)], etb)
        pltpu.sync_copy(src_h.at[s, pl.ds(g * GEDGE, GEDGE)], srcb)
        pltpu.sync_copy(dst_h.at[s, pl.ds(g * GEDGE, GEDGE)], dstb)

        def _row(i, carry2):
            for j in range(5):
                off = j * 16
                e16 = etb[i, pl.ds(off, 16)]
                en = e16 * N
                binb[g * GEDGE + i, pl.ds(off, 16)] = en + dstb[i, pl.ds(off, 16)]
                gb[g * GEDGE + i, pl.ds(off, 16)] = en + srcb[i, pl.ds(off, 16)]
            return carry2
        lax.fori_loop(0, GEDGE, _row, 0)
        return carry
    lax.fori_loop(0, NCHUNK // GEDGE, _grp, 0)

    @pl.when(c == 0)
    def _():
        pltpu.async_copy(gb, gidx_h.at[s], sem_w)

    def _hist(ch, carry):
        pltpu.sync_copy(ones, cnt.at[binb.at[ch]], add=True)
        return carry
    lax.fori_loop(0, NCHUNK, _hist, 0)
    plsc.subcore_barrier()

    start = c * 63
    end = 63 + c * 62

    def _fire_gather(k, slot):
        pltpu.async_copy(cnt.at[binb.at[k]], cbuf.at[slot], sem_g)

    _fire_gather(start, 0)

    def _scl(k, carry):
        slot = lax.rem(k - start, 2)

        @pl.when(k + 1 < end)
        def _():
            _fire_gather(k + 1, 1 - slot)
        pltpu.make_async_copy(cnt.at[binb.at[k]], cbuf.at[slot], sem_g).wait()

        @pl.when(k >= start + 2)
        def _():
            pltpu.make_async_copy(sbuf.at[slot],
                                  scale_h.at[s, pl.ds(0, CHUNK)], sem_st).wait()

        def _srow(m, carry2):
            for u in range(4):
                e = m * 4 + u
                cv = cbuf[slot, e, :]
                sbuf[slot, e, :] = one16 / jnp.maximum(cv, one16)
            return carry2
        lax.fori_loop(0, CHUNK // 4, _srow, 0)
        pltpu.async_copy(sbuf.at[slot], scale_h.at[s, pl.ds(k * CHUNK, CHUNK)],
                         sem_st)
        return carry
    lax.fori_loop(start, end, _scl, 0)
    for _ in range(2):
        pltpu.make_async_copy(sbuf.at[0], scale_h.at[s, pl.ds(0, CHUNK)],
                              sem_st).wait()

    @pl.when(c == 0)
    def _():
        pltpu.make_async_copy(gb, gidx_h.at[s], sem_w).wait()


_scale_call = functools.partial(
    pl.kernel,
    mesh=_mesh,
    out_type=[
        jax.ShapeDtypeStruct((NSUB, NCHUNK, CHUNK), jnp.int32),
        jax.ShapeDtypeStruct((NSUB, EPT, 16), jnp.float32),
    ],
    scratch_types=[
        pltpu.VMEM_SHARED((REL * N, 16), jnp.float32),
        pltpu.VMEM((GEDGE, CHUNK), jnp.int32),
        pltpu.VMEM((GEDGE, CHUNK), jnp.int32),
        pltpu.VMEM((GEDGE, CHUNK), jnp.int32),
        pltpu.VMEM((NCHUNK, CHUNK), jnp.int32),
        pltpu.VMEM((NCHUNK, CHUNK), jnp.int32),
        pltpu.VMEM((CHUNK, 16), jnp.float32),
        pltpu.VMEM((2, CHUNK, 16), jnp.float32),
        pltpu.VMEM((2, CHUNK, 16), jnp.float32),
        pltpu.VMEM((250, 16), jnp.float32),
        pltpu.SemaphoreType.DMA,
        pltpu.SemaphoreType.DMA,
        pltpu.SemaphoreType.DMA,
        pltpu.SemaphoreType.DMA,
    ],
    compiler_params=pltpu.CompilerParams(use_tc_tiling_on_sc=False),
)(_scale_body)


def _msg_body(xw_h, gidx_h, dst_h, scale_h, out_h,
              acc, gidxb, dstb, rows, sbuf, zb, sem_g, sem_sc, sem_a, sem_b):
    c = lax.axis_index("c")
    s = lax.axis_index("s")
    z16 = jnp.zeros((16,), jnp.float32)

    def _zrow(i, carry):
        for j in range(8):
            zb[i, pl.ds(j * 16, 16)] = z16
        return carry
    lax.fori_loop(0, 25, _zrow, 0)

    def _zcp(k, carry):
        pltpu.sync_copy(zb, acc.at[pl.ds(s * ROWS_PT + k * 25, 25)])
        return carry
    lax.fori_loop(0, 25, _zcp, 0)
    plsc.subcore_barrier()

    pltpu.sync_copy(gidx_h.at[s], gidxb)
    xwc = xw_h.at[c]

    def _fire_loads(ch):
        @pl.when(lax.rem(ch, 2) == 0)
        def _():
            pltpu.async_copy(dst_h.at[s, ch], dstb.at[lax.rem(ch, 4)], sem_a)
            pltpu.async_copy(scale_h.at[s, pl.ds(ch * CHUNK, CHUNK)],
                             sbuf.at[lax.rem(ch, 3)], sem_a)

        @pl.when(lax.rem(ch, 2) == 1)
        def _():
            pltpu.async_copy(dst_h.at[s, ch], dstb.at[lax.rem(ch, 4)], sem_b)
            pltpu.async_copy(scale_h.at[s, pl.ds(ch * CHUNK, CHUNK)],
                             sbuf.at[lax.rem(ch, 3)], sem_b)

    def _wait_loads(ch):
        @pl.when(lax.rem(ch, 2) == 0)
        def _():
            pltpu.make_async_copy(dst_h.at[s, ch], dstb.at[lax.rem(ch, 4)],
                                  sem_a).wait()
            pltpu.make_async_copy(scale_h.at[s, pl.ds(ch * CHUNK, CHUNK)],
                                  sbuf.at[lax.rem(ch, 3)], sem_a).wait()

        @pl.when(lax.rem(ch, 2) == 1)
        def _():
            pltpu.make_async_copy(dst_h.at[s, ch], dstb.at[lax.rem(ch, 4)],
                                  sem_b).wait()
            pltpu.make_async_copy(scale_h.at[s, pl.ds(ch * CHUNK, CHUNK)],
                                  sbuf.at[lax.rem(ch, 3)], sem_b).wait()

    def _fire_gather(ch):
        pltpu.async_copy(xwc.at[gidxb.at[ch]], rows.at[lax.rem(ch, 3)], sem_g)

    def _wait_one_scatter():
        pltpu.make_async_copy(rows.at[0], acc.at[dstb.at[0]], sem_sc).wait()

    _fire_loads(0)
    _fire_loads(1)
    _fire_gather(0)

    def _chunk(ch, carry):
        p3 = lax.rem(ch, 3)
        p4 = lax.rem(ch, 4)

        @pl.when(ch >= 2)
        def _():
            _wait_one_scatter()

        @pl.when(ch + 1 < NCHUNK)
        def _():
            _fire_gather(ch + 1)

        _wait_loads(ch)
        pltpu.make_async_copy(xwc.at[gidxb.at[ch]], rows.at[p3], sem_g).wait()

        def _mul(m, carry2):
            for u in range(8):
                e = m * 8 + u
                sv = sbuf[p3, e, :]
                for j in range(8):
                    rows[p3, e, pl.ds(j * 16, 16)] = (
                        rows[p3, e, pl.ds(j * 16, 16)] * sv)
            return carry2
        lax.fori_loop(0, CHUNK // 8, _mul, 0)
        pltpu.async_copy(rows.at[p3], acc.at[dstb.at[p4]], sem_sc, add=True)

        @pl.when(ch + 2 < NCHUNK)
        def _():
            _fire_loads(ch + 2)
        return carry
    lax.fori_loop(0, NCHUNK, _chunk, 0)
    _wait_one_scatter()
    _wait_one_scatter()
    plsc.subcore_barrier()

    def _ocp(k, carry):
        pltpu.sync_copy(acc.at[pl.ds(s * ROWS_PT + k * 25, 25)], zb)
        pltpu.sync_copy(zb, out_h.at[c, pl.ds(s * ROWS_PT + k * 25, 25)])
        return carry
    lax.fori_loop(0, 25, _ocp, 0)


_msg_call = functools.partial(
    pl.kernel,
    mesh=_mesh,
    out_type=jax.ShapeDtypeStruct((2, N, HALF), jnp.float32),
    scratch_types=[
        pltpu.VMEM_SHARED((N, HALF), jnp.float32),
        pltpu.VMEM((NCHUNK, CHUNK), jnp.int32),
        pltpu.VMEM((4, CHUNK), jnp.int32),
        pltpu.VMEM((3, CHUNK, HALF), jnp.float32),
        pltpu.VMEM((3, CHUNK, 16), jnp.float32),
        pltpu.VMEM((25, HALF), jnp.float32),
        pltpu.SemaphoreType.DMA,
        pltpu.SemaphoreType.DMA,
        pltpu.SemaphoreType.DMA,
        pltpu.SemaphoreType.DMA,
    ],
    compiler_params=pltpu.CompilerParams(use_tc_tiling_on_sc=False),
)(_msg_body)


NB = 10
BN = N // NB


def _mm1_body(x_ref, w_ref, root_ref, b_ref, xw_ref, xr_ref):
    x = x_ref[...]
    d = jnp.dot(x, w_ref[0], preferred_element_type=jnp.float32)
    xw_ref[0] = d[:, :HALF]
    xw_ref[1] = d[:, HALF:]

    @pl.when(pl.program_id(1) == 0)
    def _():
        xr_ref[...] = (jnp.dot(x, root_ref[...], preferred_element_type=jnp.float32)
                       + b_ref[...])


def _mm2_body(xr1_ref, msg_ref, w_ref, root_ref, b_ref, xw_ref, xr_ref):
    h2 = jnp.maximum(
        xr1_ref[...] + jnp.concatenate([msg_ref[0], msg_ref[1]], axis=1), 0.0)
    d = jnp.dot(h2, w_ref[0], preferred_element_type=jnp.float32)
    xw_ref[0] = d[:, :HALF]
    xw_ref[1] = d[:, HALF:]

    @pl.when(pl.program_id(1) == 0)
    def _():
        xr_ref[...] = (jnp.dot(h2, root_ref[...], preferred_element_type=jnp.float32)
                       + b_ref[...])


_mm1 = pl.pallas_call(
    _mm1_body,
    grid=(NB, REL),
    in_specs=[
        pl.BlockSpec((BN, D_IN), lambda i, r: (i, 0)),
        pl.BlockSpec((1, D_IN, HID), lambda i, r: (r, 0, 0)),
        pl.BlockSpec((D_IN, HID), lambda i, r: (0, 0)),
        pl.BlockSpec((1, HID), lambda i, r: (0, 0)),
    ],
    out_specs=[
        pl.BlockSpec((2, BN, HALF), lambda i, r: (0, r * NB + i, 0)),
        pl.BlockSpec((BN, HID), lambda i, r: (i, 0)),
    ],
    out_shape=[
        jax.ShapeDtypeStruct((2, REL * N, HALF), jnp.float32),
        jax.ShapeDtypeStruct((N, HID), jnp.float32),
    ],
)

_mm2 = pl.pallas_call(
    _mm2_body,
    grid=(NB, REL),
    in_specs=[
        pl.BlockSpec((BN, HID), lambda i, r: (i, 0)),
        pl.BlockSpec((2, BN, HALF), lambda i, r: (0, i, 0)),
        pl.BlockSpec((1, HID, HID), lambda i, r: (r, 0, 0)),
        pl.BlockSpec((HID, HID), lambda i, r: (0, 0)),
        pl.BlockSpec((1, HID), lambda i, r: (0, 0)),
    ],
    out_specs=[
        pl.BlockSpec((2, BN, HALF), lambda i, r: (0, r * NB + i, 0)),
        pl.BlockSpec((BN, HID), lambda i, r: (i, 0)),
    ],
    out_shape=[
        jax.ShapeDtypeStruct((2, REL * N, HALF), jnp.float32),
        jax.ShapeDtypeStruct((N, HID), jnp.float32),
    ],
)


GP = 128


def _pool_body(xr_ref, msg_ref, batch_ref, gw_ref, gb_ref, fw_ref, fb_ref, o_ref):
    h = jnp.maximum(
        xr_ref[...] + jnp.concatenate([msg_ref[0], msg_ref[1]], axis=1), 0.0)
    gate = jnp.sum(h * gw_ref[...], axis=1, keepdims=True) + gb_ref[...]
    gid = jax.lax.broadcasted_iota(jnp.int32, (N, GP), 1)
    ohb = batch_ref[...] == gid
    ohf = ohb.astype(jnp.float32)
    gmax = jnp.max(jnp.where(ohb, gate, -1e30), axis=0, keepdims=True)
    gmax_n = jnp.sum(ohf * gmax, axis=1, keepdims=True)
    e = jnp.exp(gate - gmax_n)
    denom = jnp.sum(ohf * e, axis=0, keepdims=True)
    denom_n = jnp.sum(ohf * denom, axis=1, keepdims=True)
    alpha = e / jnp.maximum(denom_n, 1e-16)
    pooled = jax.lax.dot_general(ohf, h * alpha, (((0,), (0,)), ((), ())),
                                 preferred_element_type=jnp.float32)
    o_ref[...] = (jnp.dot(pooled, fw_ref[...], preferred_element_type=jnp.float32)
                  + fb_ref[...])


_pool = pl.pallas_call(
    _pool_body,
    out_shape=jax.ShapeDtypeStruct((GP, OUT), jnp.float32),
)


def kernel(x, edge_index, edge_type, batch, W1, root1, b1, W2, root2, b2,
           gate_w, gate_b, fc_w, fc_b):
    et3d = edge_type.reshape(NSUB, NCHUNK, CHUNK)
    src3d = edge_index[0].reshape(NSUB, NCHUNK, CHUNK)
    dst3d = edge_index[1].reshape(NSUB, NCHUNK, CHUNK)
    gidx, scale = _scale_call(et3d, src3d, dst3d)

    xw1, xr1 = _mm1(x, W1, root1, b1.reshape(1, HID))
    msg1 = _msg_call(xw1, gidx, dst3d, scale)
    xw2, xr2 = _mm2(xr1, msg1, W2, root2, b2.reshape(1, HID))
    msg2 = _msg_call(xw2, gidx, dst3d, scale)

    out = _pool(xr2, msg2, batch.reshape(N, 1), gate_w.reshape(1, HID),
                gate_b.reshape(1, 1), fc_w, fc_b.reshape(1, OUT))
    return out[:G]


# speedup vs baseline: 1.0153x; 1.0153x over previous
import functools

import jax
import jax.numpy as jnp
from jax import lax
from jax.experimental import pallas as pl
from jax.experimental.pallas import tpu as pltpu
from jax.experimental.pallas import tpu_sc as plsc

N = 10000
E = 160000
D_IN = 256
HID = 256
OUT = 128
REL = 8
G = 64

NSUB = 16
HALF = 128
EPT = E // NSUB
CHUNK = 80
NCHUNK = EPT // CHUNK
ROWS_PT = N // NSUB
GEDGE = 25

_mesh = plsc.VectorSubcoreMesh(core_axis_name="c", subcore_axis_name="s")


def _scale_body(et_h, src_h, dst_h, gidx_h, scale_h,
                cnt, etb, srcb, dstb, binb, gb, ones, cbuf, sbuf, zb,
                sem_h, sem_g, sem_st, sem_w):
    c = lax.axis_index("c")
    s = lax.axis_index("s")
    z16 = jnp.zeros((16,), jnp.float32)
    one16 = jnp.full((16,), 1.0, jnp.float32)

    def _zrow(i, carry):
        zb[i, :] = z16
        return carry
    lax.fori_loop(0, 250, _zrow, 0)
    for k in range(20):
        pltpu.sync_copy(zb, cnt.at[pl.ds(s * 5000 + k * 250, 250)])

    def _orow(i, carry):
        ones[i, :] = one16
        return carry
    lax.fori_loop(0, CHUNK, _orow, 0)
    plsc.subcore_barrier()

    def _grp(g, carry):
        pltpu.sync_copy(et_h.at[s, pl.ds(g * GEDGE, GEDGE)], etb)
        pltpu.sync_copy(src_h.at[s, pl.ds(g * GEDGE, GEDGE)], srcb)
        pltpu.sync_copy(dst_h.at[s, pl.ds(g * GEDGE, GEDGE)], dstb)

        def _row(i, carry2):
            for j in range(5):
                off = j * 16
                e16 = etb[i, pl.ds(off, 16)]
                en = e16 * N
                binb[g * GEDGE + i, pl.ds(off, 16)] = en + dstb[i, pl.ds(off, 16)]
                gb[g * GEDGE + i, pl.ds(off, 16)] = en + srcb[i, pl.ds(off, 16)]
            return carry2
        lax.fori_loop(0, GEDGE, _row, 0)
        return carry
    lax.fori_loop(0, NCHUNK // GEDGE, _grp, 0)

    @pl.when(c == 0)
    def _():
        pltpu.async_copy(gb, gidx_h.at[s], sem_w)

    def _hist(ch, carry):
        pltpu.async_copy(ones, cnt.at[binb.at[ch]], sem_h, add=True)
        return carry
    lax.fori_loop(0, NCHUNK, _hist, 0)

    def _hdrain(ch, carry):
        pltpu.make_async_copy(ones, cnt.at[binb.at[0]], sem_h).wait()
        return carry
    lax.fori_loop(0, NCHUNK, _hdrain, 0)
    plsc.subcore_barrier()

    start = c * 63
    end = 63 + c * 62

    def _fire_gather(k, slot):
        pltpu.async_copy(cnt.at[binb.at[k]], cbuf.at[slot], sem_g)

    _fire_gather(start, 0)

    def _scl(k, carry):
        slot = lax.rem(k - start, 2)

        @pl.when(k + 1 < end)
        def _():
            _fire_gather(k + 1, 1 - slot)
        pltpu.make_async_copy(cnt.at[binb.at[k]], cbuf.at[slot], sem_g).wait()

        @pl.when(k >= start + 2)
        def _():
            pltpu.make_async_copy(sbuf.at[slot],
                                  scale_h.at[s, pl.ds(0, CHUNK)], sem_st).wait()

        def _srow(m, carry2):
            for u in range(4):
                e = m * 4 + u
                cv = cbuf[slot, e, :]
                sbuf[slot, e, :] = one16 / jnp.maximum(cv, one16)
            return carry2
        lax.fori_loop(0, CHUNK // 4, _srow, 0)
        pltpu.async_copy(sbuf.at[slot], scale_h.at[s, pl.ds(k * CHUNK, CHUNK)],
                         sem_st)
        return carry
    lax.fori_loop(start, end, _scl, 0)
    for _ in range(2):
        pltpu.make_async_copy(sbuf.at[0], scale_h.at[s, pl.ds(0, CHUNK)],
                              sem_st).wait()

    @pl.when(c == 0)
    def _():
        pltpu.make_async_copy(gb, gidx_h.at[s], sem_w).wait()


_scale_call = functools.partial(
    pl.kernel,
    mesh=_mesh,
    out_type=[
        jax.ShapeDtypeStruct((NSUB, NCHUNK, CHUNK), jnp.int32),
        jax.ShapeDtypeStruct((NSUB, EPT, 16), jnp.float32),
    ],
    scratch_types=[
        pltpu.VMEM_SHARED((REL * N, 16), jnp.float32),
        pltpu.VMEM((GEDGE, CHUNK), jnp.int32),
        pltpu.VMEM((GEDGE, CHUNK), jnp.int32),
        pltpu.VMEM((GEDGE, CHUNK), jnp.int32),
        pltpu.VMEM((NCHUNK, CHUNK), jnp.int32),
        pltpu.VMEM((NCHUNK, CHUNK), jnp.int32),
        pltpu.VMEM((CHUNK, 16), jnp.float32),
        pltpu.VMEM((2, CHUNK, 16), jnp.float32),
        pltpu.VMEM((2, CHUNK, 16), jnp.float32),
        pltpu.VMEM((250, 16), jnp.float32),
        pltpu.SemaphoreType.DMA,
        pltpu.SemaphoreType.DMA,
        pltpu.SemaphoreType.DMA,
        pltpu.SemaphoreType.DMA,
    ],
    compiler_params=pltpu.CompilerParams(use_tc_tiling_on_sc=False),
)(_scale_body)


def _msg_body(xw_h, gidx_h, dst_h, scale_h, out_h,
              acc, gidxb, dstb, rows, sbuf, zb, sem_g, sem_sc, sem_a, sem_b):
    c = lax.axis_index("c")
    s = lax.axis_index("s")
    z16 = jnp.zeros((16,), jnp.float32)

    def _zrow(i, carry):
        for j in range(8):
            zb[i, pl.ds(j * 16, 16)] = z16
        return carry
    lax.fori_loop(0, 25, _zrow, 0)

    def _zcp(k, carry):
        pltpu.sync_copy(zb, acc.at[pl.ds(s * ROWS_PT + k * 25, 25)])
        return carry
    lax.fori_loop(0, 25, _zcp, 0)
    plsc.subcore_barrier()

    pltpu.sync_copy(gidx_h.at[s], gidxb)
    xwc = xw_h.at[c]

    def _fire_loads(ch):
        @pl.when(lax.rem(ch, 2) == 0)
        def _():
            pltpu.async_copy(dst_h.at[s, ch], dstb.at[lax.rem(ch, 4)], sem_a)
            pltpu.async_copy(scale_h.at[s, pl.ds(ch * CHUNK, CHUNK)],
                             sbuf.at[lax.rem(ch, 3)], sem_a)

        @pl.when(lax.rem(ch, 2) == 1)
        def _():
            pltpu.async_copy(dst_h.at[s, ch], dstb.at[lax.rem(ch, 4)], sem_b)
            pltpu.async_copy(scale_h.at[s, pl.ds(ch * CHUNK, CHUNK)],
                             sbuf.at[lax.rem(ch, 3)], sem_b)

    def _wait_loads(ch):
        @pl.when(lax.rem(ch, 2) == 0)
        def _():
            pltpu.make_async_copy(dst_h.at[s, ch], dstb.at[lax.rem(ch, 4)],
                                  sem_a).wait()
            pltpu.make_async_copy(scale_h.at[s, pl.ds(ch * CHUNK, CHUNK)],
                                  sbuf.at[lax.rem(ch, 3)], sem_a).wait()

        @pl.when(lax.rem(ch, 2) == 1)
        def _():
            pltpu.make_async_copy(dst_h.at[s, ch], dstb.at[lax.rem(ch, 4)],
                                  sem_b).wait()
            pltpu.make_async_copy(scale_h.at[s, pl.ds(ch * CHUNK, CHUNK)],
                                  sbuf.at[lax.rem(ch, 3)], sem_b).wait()

    def _fire_gather(ch):
        pltpu.async_copy(xwc.at[gidxb.at[ch]], rows.at[lax.rem(ch, 3)], sem_g)

    def _wait_one_scatter():
        pltpu.make_async_copy(rows.at[0], acc.at[dstb.at[0]], sem_sc).wait()

    _fire_loads(0)
    _fire_loads(1)
    _fire_gather(0)

    def _chunk(ch, carry):
        p3 = lax.rem(ch, 3)
        p4 = lax.rem(ch, 4)

        @pl.when(ch >= 2)
        def _():
            _wait_one_scatter()

        @pl.when(ch + 1 < NCHUNK)
        def _():
            _fire_gather(ch + 1)

        _wait_loads(ch)
        pltpu.make_async_copy(xwc.at[gidxb.at[ch]], rows.at[p3], sem_g).wait()

        def _mul(m, carry2):
            for u in range(8):
                e = m * 8 + u
                sv = sbuf[p3, e, :]
                for j in range(8):
                    rows[p3, e, pl.ds(j * 16, 16)] = (
                        rows[p3, e, pl.ds(j * 16, 16)] * sv)
            return carry2
        lax.fori_loop(0, CHUNK // 8, _mul, 0)
        pltpu.async_copy(rows.at[p3], acc.at[dstb.at[p4]], sem_sc, add=True)

        @pl.when(ch + 2 < NCHUNK)
        def _():
            _fire_loads(ch + 2)
        return carry
    lax.fori_loop(0, NCHUNK, _chunk, 0)
    _wait_one_scatter()
    _wait_one_scatter()
    plsc.subcore_barrier()

    def _ocp(k, carry):
        pltpu.sync_copy(acc.at[pl.ds(s * ROWS_PT + k * 25, 25)], zb)
        pltpu.sync_copy(zb, out_h.at[c, pl.ds(s * ROWS_PT + k * 25, 25)])
        return carry
    lax.fori_loop(0, 25, _ocp, 0)


_msg_call = functools.partial(
    pl.kernel,
    mesh=_mesh,
    out_type=jax.ShapeDtypeStruct((2, N, HALF), jnp.float32),
    scratch_types=[
        pltpu.VMEM_SHARED((N, HALF), jnp.float32),
        pltpu.VMEM((NCHUNK, CHUNK), jnp.int32),
        pltpu.VMEM((4, CHUNK), jnp.int32),
        pltpu.VMEM((3, CHUNK, HALF), jnp.float32),
        pltpu.VMEM((3, CHUNK, 16), jnp.float32),
        pltpu.VMEM((25, HALF), jnp.float32),
        pltpu.SemaphoreType.DMA,
        pltpu.SemaphoreType.DMA,
        pltpu.SemaphoreType.DMA,
        pltpu.SemaphoreType.DMA,
    ],
    compiler_params=pltpu.CompilerParams(use_tc_tiling_on_sc=False),
)(_msg_body)


NB = 10
BN = N // NB


def _mm1_body(x_ref, w_ref, root_ref, b_ref, xw_ref, xr_ref):
    x = x_ref[...]
    d = jnp.dot(x, w_ref[0], preferred_element_type=jnp.float32)
    xw_ref[0] = d[:, :HALF]
    xw_ref[1] = d[:, HALF:]

    @pl.when(pl.program_id(1) == 0)
    def _():
        xr_ref[...] = (jnp.dot(x, root_ref[...], preferred_element_type=jnp.float32)
                       + b_ref[...])


def _mm2_body(xr1_ref, msg_ref, w_ref, root_ref, b_ref, xw_ref, xr_ref):
    h2 = jnp.maximum(
        xr1_ref[...] + jnp.concatenate([msg_ref[0], msg_ref[1]], axis=1), 0.0)
    d = jnp.dot(h2, w_ref[0], preferred_element_type=jnp.float32)
    xw_ref[0] = d[:, :HALF]
    xw_ref[1] = d[:, HALF:]

    @pl.when(pl.program_id(1) == 0)
    def _():
        xr_ref[...] = (jnp.dot(h2, root_ref[...], preferred_element_type=jnp.float32)
                       + b_ref[...])


_mm1 = pl.pallas_call(
    _mm1_body,
    grid=(NB, REL),
    in_specs=[
        pl.BlockSpec((BN, D_IN), lambda i, r: (i, 0)),
        pl.BlockSpec((1, D_IN, HID), lambda i, r: (r, 0, 0)),
        pl.BlockSpec((D_IN, HID), lambda i, r: (0, 0)),
        pl.BlockSpec((1, HID), lambda i, r: (0, 0)),
    ],
    out_specs=[
        pl.BlockSpec((2, BN, HALF), lambda i, r: (0, r * NB + i, 0)),
        pl.BlockSpec((BN, HID), lambda i, r: (i, 0)),
    ],
    out_shape=[
        jax.ShapeDtypeStruct((2, REL * N, HALF), jnp.float32),
        jax.ShapeDtypeStruct((N, HID), jnp.float32),
    ],
)

_mm2 = pl.pallas_call(
    _mm2_body,
    grid=(NB, REL),
    in_specs=[
        pl.BlockSpec((BN, HID), lambda i, r: (i, 0)),
        pl.BlockSpec((2, BN, HALF), lambda i, r: (0, i, 0)),
        pl.BlockSpec((1, HID, HID), lambda i, r: (r, 0, 0)),
        pl.BlockSpec((HID, HID), lambda i, r: (0, 0)),
        pl.BlockSpec((1, HID), lambda i, r: (0, 0)),
    ],
    out_specs=[
        pl.BlockSpec((2, BN, HALF), lambda i, r: (0, r * NB + i, 0)),
        pl.BlockSpec((BN, HID), lambda i, r: (i, 0)),
    ],
    out_shape=[
        jax.ShapeDtypeStruct((2, REL * N, HALF), jnp.float32),
        jax.ShapeDtypeStruct((N, HID), jnp.float32),
    ],
)


GP = 128


def _pool_body(xr_ref, msg_ref, batch_ref, gw_ref, gb_ref, fw_ref, fb_ref, o_ref):
    h = jnp.maximum(
        xr_ref[...] + jnp.concatenate([msg_ref[0], msg_ref[1]], axis=1), 0.0)
    gate = jnp.sum(h * gw_ref[...], axis=1, keepdims=True) + gb_ref[...]
    gid = jax.lax.broadcasted_iota(jnp.int32, (N, GP), 1)
    ohb = batch_ref[...] == gid
    ohf = ohb.astype(jnp.float32)
    gmax = jnp.max(jnp.where(ohb, gate, -1e30), axis=0, keepdims=True)
    gmax_n = jnp.sum(ohf * gmax, axis=1, keepdims=True)
    e = jnp.exp(gate - gmax_n)
    denom = jnp.sum(ohf * e, axis=0, keepdims=True)
    denom_n = jnp.sum(ohf * denom, axis=1, keepdims=True)
    alpha = e / jnp.maximum(denom_n, 1e-16)
    pooled = jax.lax.dot_general(ohf, h * alpha, (((0,), (0,)), ((), ())),
                                 preferred_element_type=jnp.float32)
    o_ref[...] = (jnp.dot(pooled, fw_ref[...], preferred_element_type=jnp.float32)
                  + fb_ref[...])


_pool = pl.pallas_call(
    _pool_body,
    out_shape=jax.ShapeDtypeStruct((GP, OUT), jnp.float32),
)


def kernel(x, edge_index, edge_type, batch, W1, root1, b1, W2, root2, b2,
           gate_w, gate_b, fc_w, fc_b):
    et3d = edge_type.reshape(NSUB, NCHUNK, CHUNK)
    src3d = edge_index[0].reshape(NSUB, NCHUNK, CHUNK)
    dst3d = edge_index[1].reshape(NSUB, NCHUNK, CHUNK)
    gidx, scale = _scale_call(et3d, src3d, dst3d)

    xw1, xr1 = _mm1(x, W1, root1, b1.reshape(1, HID))
    msg1 = _msg_call(xw1, gidx, dst3d, scale)
    xw2, xr2 = _mm2(xr1, msg1, W2, root2, b2.reshape(1, HID))
    msg2 = _msg_call(xw2, gidx, dst3d, scale)

    out = _pool(xr2, msg2, batch.reshape(N, 1), gate_w.reshape(1, HID),
                gate_b.reshape(1, 1), fc_w, fc_b.reshape(1, OUT))
    return out[:G]


# speedup vs baseline: 1.0900x; 1.0735x over previous
import functools

import jax
import jax.numpy as jnp
from jax import lax
from jax.experimental import pallas as pl
from jax.experimental.pallas import tpu as pltpu
from jax.experimental.pallas import tpu_sc as plsc

N = 10000
E = 160000
D_IN = 256
HID = 256
OUT = 128
REL = 8
G = 64

NSUB = 16
HALF = 128
EPT = E // NSUB
CHUNK = 80
NCHUNK = EPT // CHUNK
ROWS_PT = N // NSUB
GEDGE = 25

_mesh = plsc.VectorSubcoreMesh(core_axis_name="c", subcore_axis_name="s")


def _scale_body(et_h, src_h, dst_h, gidx_h, scale_h,
                cnt, etb, srcb, dstb, binb, gb, ones, cbuf, sbuf, zb,
                sem_h, sem_g, sem_st, sem_w):
    c = lax.axis_index("c")
    s = lax.axis_index("s")
    z16 = jnp.zeros((16,), jnp.float32)
    one16 = jnp.full((16,), 1.0, jnp.float32)

    def _zrow(i, carry):
        zb[i, :] = z16
        return carry
    lax.fori_loop(0, 250, _zrow, 0)
    for k in range(20):
        pltpu.sync_copy(zb, cnt.at[pl.ds(s * 5000 + k * 250, 250)])

    def _orow(i, carry):
        ones[i, :] = one16
        return carry
    lax.fori_loop(0, CHUNK, _orow, 0)
    plsc.subcore_barrier()

    def _grp(g, carry):
        pltpu.sync_copy(et_h.at[s, pl.ds(g * GEDGE, GEDGE)], etb)
        pltpu.sync_copy(src_h.at[s, pl.ds(g * GEDGE, GEDGE)], srcb)
        pltpu.sync_copy(dst_h.at[s, pl.ds(g * GEDGE, GEDGE)], dstb)

        def _row(i, carry2):
            for j in range(5):
                off = j * 16
                e16 = etb[i, pl.ds(off, 16)]
                en = e16 * N
                binb[g * GEDGE + i, pl.ds(off, 16)] = en + dstb[i, pl.ds(off, 16)]
                gb[g * GEDGE + i, pl.ds(off, 16)] = en + srcb[i, pl.ds(off, 16)]
            return carry2
        lax.fori_loop(0, GEDGE, _row, 0)
        return carry
    lax.fori_loop(0, NCHUNK // GEDGE, _grp, 0)

    @pl.when(c == 0)
    def _():
        pltpu.async_copy(gb, gidx_h.at[s], sem_w)

    def _hist(ch, carry):
        pltpu.async_copy(ones, cnt.at[binb.at[ch]], sem_h, add=True)
        return carry
    lax.fori_loop(0, NCHUNK, _hist, 0)

    def _hdrain(ch, carry):
        pltpu.make_async_copy(ones, cnt.at[binb.at[0]], sem_h).wait()
        return carry
    lax.fori_loop(0, NCHUNK, _hdrain, 0)
    plsc.subcore_barrier()

    start = c * 63
    end = 63 + c * 62

    def _fire_gather(k, slot):
        pltpu.async_copy(cnt.at[binb.at[k]], cbuf.at[slot], sem_g)

    _fire_gather(start, 0)

    def _scl(k, carry):
        slot = lax.rem(k - start, 2)

        @pl.when(k + 1 < end)
        def _():
            _fire_gather(k + 1, 1 - slot)
        pltpu.make_async_copy(cnt.at[binb.at[k]], cbuf.at[slot], sem_g).wait()

        @pl.when(k >= start + 2)
        def _():
            pltpu.make_async_copy(sbuf.at[slot],
                                  scale_h.at[s, pl.ds(0, CHUNK)], sem_st).wait()

        def _srow(m, carry2):
            for u in range(4):
                e = m * 4 + u
                cv = cbuf[slot, e, :]
                sbuf[slot, e, :] = one16 / jnp.maximum(cv, one16)
            return carry2
        lax.fori_loop(0, CHUNK // 4, _srow, 0)
        pltpu.async_copy(sbuf.at[slot], scale_h.at[s, pl.ds(k * CHUNK, CHUNK)],
                         sem_st)
        return carry
    lax.fori_loop(start, end, _scl, 0)
    for _ in range(2):
        pltpu.make_async_copy(sbuf.at[0], scale_h.at[s, pl.ds(0, CHUNK)],
                              sem_st).wait()

    @pl.when(c == 0)
    def _():
        pltpu.make_async_copy(gb, gidx_h.at[s], sem_w).wait()


_scale_call = functools.partial(
    pl.kernel,
    mesh=_mesh,
    out_type=[
        jax.ShapeDtypeStruct((NSUB, NCHUNK, CHUNK), jnp.int32),
        jax.ShapeDtypeStruct((NSUB, EPT, 16), jnp.float32),
    ],
    scratch_types=[
        pltpu.VMEM_SHARED((REL * N, 16), jnp.float32),
        pltpu.VMEM((GEDGE, CHUNK), jnp.int32),
        pltpu.VMEM((GEDGE, CHUNK), jnp.int32),
        pltpu.VMEM((GEDGE, CHUNK), jnp.int32),
        pltpu.VMEM((NCHUNK, CHUNK), jnp.int32),
        pltpu.VMEM((NCHUNK, CHUNK), jnp.int32),
        pltpu.VMEM((CHUNK, 16), jnp.float32),
        pltpu.VMEM((2, CHUNK, 16), jnp.float32),
        pltpu.VMEM((2, CHUNK, 16), jnp.float32),
        pltpu.VMEM((250, 16), jnp.float32),
        pltpu.SemaphoreType.DMA,
        pltpu.SemaphoreType.DMA,
        pltpu.SemaphoreType.DMA,
        pltpu.SemaphoreType.DMA,
    ],
    compiler_params=pltpu.CompilerParams(use_tc_tiling_on_sc=False),
)(_scale_body)


def _msg_body(xw_h, gidx_h, dst_h, scale_h, out_h,
              acc, gidxb, dstb, rows, sbuf, zb, sem_g, sem_sc, sem_a, sem_b):
    c = lax.axis_index("c")
    s = lax.axis_index("s")
    z16 = jnp.zeros((16,), jnp.float32)

    def _zrow(i, carry):
        for j in range(8):
            zb[i, pl.ds(j * 16, 16)] = z16
        return carry
    lax.fori_loop(0, 25, _zrow, 0)

    def _zcp(k, carry):
        pltpu.sync_copy(zb, acc.at[pl.ds(s * ROWS_PT + k * 25, 25)])
        return carry
    lax.fori_loop(0, 25, _zcp, 0)
    plsc.subcore_barrier()

    pltpu.sync_copy(gidx_h.at[s], gidxb)
    xwc = xw_h.at[c]

    def _fire_loads(ch):
        @pl.when(lax.rem(ch, 2) == 0)
        def _():
            pltpu.async_copy(dst_h.at[s, ch], dstb.at[lax.rem(ch, 4)], sem_a)
            pltpu.async_copy(scale_h.at[s, pl.ds(ch * CHUNK, CHUNK)],
                             sbuf.at[lax.rem(ch, 3)], sem_a)

        @pl.when(lax.rem(ch, 2) == 1)
        def _():
            pltpu.async_copy(dst_h.at[s, ch], dstb.at[lax.rem(ch, 4)], sem_b)
            pltpu.async_copy(scale_h.at[s, pl.ds(ch * CHUNK, CHUNK)],
                             sbuf.at[lax.rem(ch, 3)], sem_b)

    def _wait_loads(ch):
        @pl.when(lax.rem(ch, 2) == 0)
        def _():
            pltpu.make_async_copy(dst_h.at[s, ch], dstb.at[lax.rem(ch, 4)],
                                  sem_a).wait()
            pltpu.make_async_copy(scale_h.at[s, pl.ds(ch * CHUNK, CHUNK)],
                                  sbuf.at[lax.rem(ch, 3)], sem_a).wait()

        @pl.when(lax.rem(ch, 2) == 1)
        def _():
            pltpu.make_async_copy(dst_h.at[s, ch], dstb.at[lax.rem(ch, 4)],
                                  sem_b).wait()
            pltpu.make_async_copy(scale_h.at[s, pl.ds(ch * CHUNK, CHUNK)],
                                  sbuf.at[lax.rem(ch, 3)], sem_b).wait()

    def _fire_gather(ch):
        pltpu.async_copy(xwc.at[gidxb.at[ch]], rows.at[lax.rem(ch, 3)], sem_g)

    def _wait_one_scatter():
        pltpu.make_async_copy(rows.at[0], acc.at[dstb.at[0]], sem_sc).wait()

    _fire_loads(0)
    _fire_loads(1)
    _fire_gather(0)

    def _chunk(ch, carry):
        p3 = lax.rem(ch, 3)
        p4 = lax.rem(ch, 4)

        @pl.when(ch >= 2)
        def _():
            _wait_one_scatter()

        @pl.when(ch + 1 < NCHUNK)
        def _():
            _fire_gather(ch + 1)

        _wait_loads(ch)
        pltpu.make_async_copy(xwc.at[gidxb.at[ch]], rows.at[p3], sem_g).wait()

        def _mul(m, carry2):
            for u in range(8):
                e = m * 8 + u
                sv = sbuf[p3, e, :]
                for j in range(8):
                    rows[p3, e, pl.ds(j * 16, 16)] = (
                        rows[p3, e, pl.ds(j * 16, 16)] * sv)
            return carry2
        lax.fori_loop(0, CHUNK // 8, _mul, 0)
        pltpu.async_copy(rows.at[p3], acc.at[dstb.at[p4]], sem_sc, add=True)

        @pl.when(ch + 2 < NCHUNK)
        def _():
            _fire_loads(ch + 2)
        return carry
    lax.fori_loop(0, NCHUNK, _chunk, 0)
    _wait_one_scatter()
    _wait_one_scatter()
    plsc.subcore_barrier()

    def _ocp(k, carry):
        pltpu.sync_copy(acc.at[pl.ds(s * ROWS_PT + k * 25, 25)], zb)
        pltpu.sync_copy(zb, out_h.at[c, pl.ds(s * ROWS_PT + k * 25, 25)])
        return carry
    lax.fori_loop(0, 25, _ocp, 0)


_msg_call = functools.partial(
    pl.kernel,
    mesh=_mesh,
    out_type=jax.ShapeDtypeStruct((2, N, HALF), jnp.float32),
    scratch_types=[
        pltpu.VMEM_SHARED((N, HALF), jnp.float32),
        pltpu.VMEM((NCHUNK, CHUNK), jnp.int32),
        pltpu.VMEM((4, CHUNK), jnp.int32),
        pltpu.VMEM((3, CHUNK, HALF), jnp.float32),
        pltpu.VMEM((3, CHUNK, 16), jnp.float32),
        pltpu.VMEM((25, HALF), jnp.float32),
        pltpu.SemaphoreType.DMA,
        pltpu.SemaphoreType.DMA,
        pltpu.SemaphoreType.DMA,
        pltpu.SemaphoreType.DMA,
    ],
    compiler_params=pltpu.CompilerParams(use_tc_tiling_on_sc=False),
)(_msg_body)


NB = 2
BN = N // NB


def _mm1_body(x_ref, w_ref, root_ref, b_ref, xw_ref, xr_ref):
    x = x_ref[...]
    d = jnp.dot(x, w_ref[0], preferred_element_type=jnp.float32)
    xw_ref[0] = d[:, :HALF]
    xw_ref[1] = d[:, HALF:]

    @pl.when(pl.program_id(1) == 0)
    def _():
        xr_ref[...] = (jnp.dot(x, root_ref[...], preferred_element_type=jnp.float32)
                       + b_ref[...])


def _mm2_body(xr1_ref, msg_ref, w_ref, root_ref, b_ref, xw_ref, xr_ref):
    h2 = jnp.maximum(
        xr1_ref[...] + jnp.concatenate([msg_ref[0], msg_ref[1]], axis=1), 0.0)
    d = jnp.dot(h2, w_ref[0], preferred_element_type=jnp.float32)
    xw_ref[0] = d[:, :HALF]
    xw_ref[1] = d[:, HALF:]

    @pl.when(pl.program_id(1) == 0)
    def _():
        xr_ref[...] = (jnp.dot(h2, root_ref[...], preferred_element_type=jnp.float32)
                       + b_ref[...])


_mm1 = pl.pallas_call(
    _mm1_body,
    grid=(NB, REL),
    in_specs=[
        pl.BlockSpec((BN, D_IN), lambda i, r: (i, 0)),
        pl.BlockSpec((1, D_IN, HID), lambda i, r: (r, 0, 0)),
        pl.BlockSpec((D_IN, HID), lambda i, r: (0, 0)),
        pl.BlockSpec((1, HID), lambda i, r: (0, 0)),
    ],
    out_specs=[
        pl.BlockSpec((2, BN, HALF), lambda i, r: (0, r * NB + i, 0)),
        pl.BlockSpec((BN, HID), lambda i, r: (i, 0)),
    ],
    out_shape=[
        jax.ShapeDtypeStruct((2, REL * N, HALF), jnp.float32),
        jax.ShapeDtypeStruct((N, HID), jnp.float32),
    ],
)

_mm2 = pl.pallas_call(
    _mm2_body,
    grid=(NB, REL),
    in_specs=[
        pl.BlockSpec((BN, HID), lambda i, r: (i, 0)),
        pl.BlockSpec((2, BN, HALF), lambda i, r: (0, i, 0)),
        pl.BlockSpec((1, HID, HID), lambda i, r: (r, 0, 0)),
        pl.BlockSpec((HID, HID), lambda i, r: (0, 0)),
        pl.BlockSpec((1, HID), lambda i, r: (0, 0)),
    ],
    out_specs=[
        pl.BlockSpec((2, BN, HALF), lambda i, r: (0, r * NB + i, 0)),
        pl.BlockSpec((BN, HID), lambda i, r: (i, 0)),
    ],
    out_shape=[
        jax.ShapeDtypeStruct((2, REL * N, HALF), jnp.float32),
        jax.ShapeDtypeStruct((N, HID), jnp.float32),
    ],
)


GP = 128


def _pool_body(xr_ref, msg_ref, batch_ref, gw_ref, gb_ref, fw_ref, fb_ref, o_ref):
    h = jnp.maximum(
        xr_ref[...] + jnp.concatenate([msg_ref[0], msg_ref[1]], axis=1), 0.0)
    gate = jnp.sum(h * gw_ref[...], axis=1, keepdims=True) + gb_ref[...]
    gid = jax.lax.broadcasted_iota(jnp.int32, (N, GP), 1)
    ohb = batch_ref[...] == gid
    ohf = ohb.astype(jnp.float32)
    gmax = jnp.max(jnp.where(ohb, gate, -1e30), axis=0, keepdims=True)
    gmax_n = jnp.sum(ohf * gmax, axis=1, keepdims=True)
    e = jnp.exp(gate - gmax_n)
    denom = jnp.sum(ohf * e, axis=0, keepdims=True)
    denom_n = jnp.sum(ohf * denom, axis=1, keepdims=True)
    alpha = e / jnp.maximum(denom_n, 1e-16)
    pooled = jax.lax.dot_general(ohf, h * alpha, (((0,), (0,)), ((), ())),
                                 preferred_element_type=jnp.float32)
    o_ref[...] = (jnp.dot(pooled, fw_ref[...], preferred_element_type=jnp.float32)
                  + fb_ref[...])


_pool = pl.pallas_call(
    _pool_body,
    out_shape=jax.ShapeDtypeStruct((GP, OUT), jnp.float32),
)


def kernel(x, edge_index, edge_type, batch, W1, root1, b1, W2, root2, b2,
           gate_w, gate_b, fc_w, fc_b):
    et3d = edge_type.reshape(NSUB, NCHUNK, CHUNK)
    src3d = edge_index[0].reshape(NSUB, NCHUNK, CHUNK)
    dst3d = edge_index[1].reshape(NSUB, NCHUNK, CHUNK)
    gidx, scale = _scale_call(et3d, src3d, dst3d)

    xw1, xr1 = _mm1(x, W1, root1, b1.reshape(1, HID))
    msg1 = _msg_call(xw1, gidx, dst3d, scale)
    xw2, xr2 = _mm2(xr1, msg1, W2, root2, b2.reshape(1, HID))
    msg2 = _msg_call(xw2, gidx, dst3d, scale)

    out = _pool(xr2, msg2, batch.reshape(N, 1), gate_w.reshape(1, HID),
                gate_b.reshape(1, 1), fc_w, fc_b.reshape(1, OUT))
    return out[:G]
